# Initial kernel scaffold; baseline (speedup 1.0000x reference)
import jax
import jax.numpy as jnp
from jax import lax
from jax.experimental import pallas as pl
from jax.experimental.pallas import tpu as pltpu
from jax.experimental.pallas import tpu_sc as plsc

B = 16384
F = 26
D = 16
N_ROWS = B * F
NC, NS = 2, 16
NW = NC * NS
ROWS_PER_W = N_ROWS // NW
CHUNK = 1024
SUB = 128
N_CHUNKS = ROWS_PER_W // CHUNK
N_SUB = CHUNK // SUB

BM = 512


def _sc_gather_body(idx_hbm, table_hbm, out_hbm, idx_v, rows_v, sem):
    c = lax.axis_index("c")
    s = lax.axis_index("s")
    wid = s * NC + c
    base = wid * ROWS_PER_W
    pltpu.sync_copy(idx_hbm.at[pl.ds(base, ROWS_PER_W)], idx_v)

    def chunk_body(ci, carry):
        row0 = ci * CHUNK
        copies = []
        for j in range(N_SUB):
            cp = pltpu.make_async_copy(
                table_hbm.at[idx_v.at[pl.ds(row0 + j * SUB, SUB)]],
                rows_v.at[pl.ds(j * SUB, SUB)],
                sem,
            )
            cp.start()
            copies.append(cp)
        for cp in copies:
            cp.wait()
        pltpu.sync_copy(rows_v, out_hbm.at[pl.ds(base + row0, CHUNK)])
        return carry

    lax.fori_loop(0, N_CHUNKS, chunk_body, 0)


_gather = pl.kernel(
    _sc_gather_body,
    out_type=jax.ShapeDtypeStruct((N_ROWS, D), jnp.float32),
    mesh=plsc.VectorSubcoreMesh(core_axis_name="c", subcore_axis_name="s"),
    scratch_types=[
        pltpu.VMEM((ROWS_PER_W,), jnp.int32),
        pltpu.VMEM((CHUNK, D), jnp.float32),
        pltpu.SemaphoreType.DMA,
    ],
)


def _mlp_body(xe, xn, w1e, w1n, b1, w2, b2, w3, b3, o):
    h = jnp.dot(xe[...], w1e[...], preferred_element_type=jnp.float32)
    h = h + jnp.dot(xn[...], w1n[...], preferred_element_type=jnp.float32)
    h = jnp.maximum(h + b1[...], 0.0)
    h = jnp.maximum(jnp.dot(h, w2[...], preferred_element_type=jnp.float32) + b2[...], 0.0)
    o[...] = jnp.dot(h, w3[...], preferred_element_type=jnp.float32) + b3[...]


def _mlp(xe, xn, w1e, w1n, b1, w2, b2, w3, b3):
    nn = xn.shape[1]
    h1 = w1e.shape[1]
    h2 = w2.shape[1]
    return pl.pallas_call(
        _mlp_body,
        grid=(B // BM,),
        in_specs=[
            pl.BlockSpec((BM, F * D), lambda i: (i, 0)),
            pl.BlockSpec((BM, nn), lambda i: (i, 0)),
            pl.BlockSpec((F * D, h1), lambda i: (0, 0)),
            pl.BlockSpec((nn, h1), lambda i: (0, 0)),
            pl.BlockSpec((1, h1), lambda i: (0, 0)),
            pl.BlockSpec((h1, h2), lambda i: (0, 0)),
            pl.BlockSpec((1, h2), lambda i: (0, 0)),
            pl.BlockSpec((h2, 1), lambda i: (0, 0)),
            pl.BlockSpec((1, 1), lambda i: (0, 0)),
        ],
        out_specs=pl.BlockSpec((BM, 1), lambda i: (i, 0)),
        out_shape=jax.ShapeDtypeStruct((B, 1), jnp.float32),
    )(xe, xn, w1e, w1n, b1, w2, b2, w3, b3)


def kernel(x_categorical, x_numerical, emb_table, W1, b1, W2, b2, W3, b3):
    idx = x_categorical.astype(jnp.int32).reshape(-1)
    embeds = _gather(idx, emb_table)
    xe = embeds.reshape(B, F * D)
    out = _mlp(
        xe,
        x_numerical,
        W1[: F * D],
        W1[F * D :],
        b1.reshape(1, -1),
        W2,
        b2.reshape(1, -1),
        W3,
        b3.reshape(1, -1),
    )
    return out.reshape(B)


# speedup vs baseline: 15.9607x; 15.9607x over previous
import jax
import jax.numpy as jnp
from jax import lax
from jax.experimental import pallas as pl
from jax.experimental.pallas import tpu as pltpu
from jax.experimental.pallas import tpu_sc as plsc

B = 16384
F = 26
D = 16
N_ROWS = B * F
NC, NS = 2, 16
NW = NC * NS
ROWS_PER_W = N_ROWS // NW
CHUNK = 1024
SUB = 128
N_CHUNKS = ROWS_PER_W // CHUNK
N_SUB = CHUNK // SUB

BM = 512


def _sc_gather_body(idx_hbm, table_hbm, out_hbm, idx_v, rows_v, sem):
    c = lax.axis_index("c")
    s = lax.axis_index("s")
    wid = s * NC + c
    base = wid * ROWS_PER_W
    pltpu.sync_copy(idx_hbm.at[pl.ds(base, ROWS_PER_W)], idx_v)

    def chunk_body(ci, carry):
        row0 = ci * CHUNK
        copies = []
        for j in range(N_SUB):
            cp = pltpu.make_async_copy(
                table_hbm.at[idx_v.at[pl.ds(row0 + j * SUB, SUB)]],
                rows_v.at[pl.ds(j * SUB, SUB)],
                sem,
            )
            cp.start()
            copies.append(cp)
        for cp in copies:
            cp.wait()
        pltpu.sync_copy(rows_v, out_hbm.at[pl.ds(base + row0, CHUNK)])
        return carry

    lax.fori_loop(0, N_CHUNKS, chunk_body, 0)


_gather = pl.kernel(
    _sc_gather_body,
    out_type=jax.ShapeDtypeStruct((N_ROWS, D), jnp.float32),
    mesh=plsc.VectorSubcoreMesh(core_axis_name="c", subcore_axis_name="s"),
    compiler_params=pltpu.CompilerParams(use_tc_tiling_on_sc=False),
    scratch_types=[
        pltpu.VMEM((ROWS_PER_W,), jnp.int32),
        pltpu.VMEM((CHUNK, D), jnp.float32),
        pltpu.SemaphoreType.DMA,
    ],
)


def _mlp_body(xe, xn, w1e, w1n, b1, w2, b2, w3, b3, o):
    h = jnp.dot(xe[...], w1e[...], preferred_element_type=jnp.float32)
    h = h + jnp.dot(xn[...], w1n[...], preferred_element_type=jnp.float32)
    h = jnp.maximum(h + b1[...], 0.0)
    h = jnp.maximum(jnp.dot(h, w2[...], preferred_element_type=jnp.float32) + b2[...], 0.0)
    o[...] = jnp.dot(h, w3[...], preferred_element_type=jnp.float32) + b3[...]


def _mlp(xe, xn, w1e, w1n, b1, w2, b2, w3, b3):
    nn = xn.shape[1]
    h1 = w1e.shape[1]
    h2 = w2.shape[1]
    return pl.pallas_call(
        _mlp_body,
        grid=(B // BM,),
        in_specs=[
            pl.BlockSpec((BM, F * D), lambda i: (i, 0)),
            pl.BlockSpec((BM, nn), lambda i: (i, 0)),
            pl.BlockSpec((F * D, h1), lambda i: (0, 0)),
            pl.BlockSpec((nn, h1), lambda i: (0, 0)),
            pl.BlockSpec((1, h1), lambda i: (0, 0)),
            pl.BlockSpec((h1, h2), lambda i: (0, 0)),
            pl.BlockSpec((1, h2), lambda i: (0, 0)),
            pl.BlockSpec((h2, 1), lambda i: (0, 0)),
            pl.BlockSpec((1, 1), lambda i: (0, 0)),
        ],
        out_specs=pl.BlockSpec((BM, 1), lambda i: (i, 0)),
        out_shape=jax.ShapeDtypeStruct((B, 1), jnp.float32),
    )(xe, xn, w1e, w1n, b1, w2, b2, w3, b3)


def kernel(x_categorical, x_numerical, emb_table, W1, b1, W2, b2, W3, b3):
    idx = x_categorical.astype(jnp.int32).reshape(-1)
    embeds = _gather(idx, emb_table)
    xe = embeds.reshape(B, F * D)
    out = _mlp(
        xe,
        x_numerical,
        W1[: F * D],
        W1[F * D :],
        b1.reshape(1, -1),
        W2,
        b2.reshape(1, -1),
        W3,
        b3.reshape(1, -1),
    )
    return out.reshape(B)
